# Initial kernel scaffold; baseline (speedup 1.0000x reference)
import jax
import jax.numpy as jnp
from jax.experimental import pallas as pl


def kernel(x, edges, rot, pos, batch, front_ptr, back_ptr, flat_new_idx, W_rel, b_rel, W_root):
    raise NotImplementedError("write your pallas kernel here")


# speedup vs baseline: 28.4472x; 28.4472x over previous
import functools

import jax
import jax.numpy as jnp
from jax import lax
from jax.experimental import pallas as pl
from jax.experimental.pallas import tpu as pltpu
from jax.experimental.pallas import tpu_sc as plsc

N = 10000
E = 320000
D = 128
NPAD = 10240
NSUB = 16
EDGES_PER_SUB = E // NSUB
EDGE_VECS = EDGES_PER_SUB // 16
NODES_PER_SUB = NPAD // NSUB
ROW_BLOCK = 640


def _tc_body(x_ref, e_ref, w1_ref, w2_ref, t_ref, rb_ref):
    res = jnp.dot(x_ref[...], w1_ref[...], preferred_element_type=jnp.float32)
    res = res + jnp.dot(e_ref[...], w2_ref[...],
                        preferred_element_type=jnp.float32)
    t_ref[...] = res[:, 0:1]
    rb_ref[...] = res[:, 1:2]


def _project(x, extra, w1, w2):
    grid = NPAD // ROW_BLOCK
    return pl.pallas_call(
        _tc_body,
        grid=(grid,),
        in_specs=[
            pl.BlockSpec((ROW_BLOCK, D), lambda i: (i, 0)),
            pl.BlockSpec((ROW_BLOCK, 8), lambda i: (i, 0)),
            pl.BlockSpec((D, 2), lambda i: (0, 0)),
            pl.BlockSpec((8, 2), lambda i: (0, 0)),
        ],
        out_specs=[
            pl.BlockSpec((ROW_BLOCK, 1), lambda i: (i, 0)),
            pl.BlockSpec((ROW_BLOCK, 1), lambda i: (i, 0)),
        ],
        out_shape=[
            jax.ShapeDtypeStruct((NPAD, 1), jnp.float32),
            jax.ShapeDtypeStruct((NPAD, 1), jnp.float32),
        ],
    )(x, extra, w1, w2)


def _sc_scatter(t, rb, src, dst):
    mesh = plsc.VectorSubcoreMesh(
        core_axis_name="c", subcore_axis_name="s", num_cores=1)

    @functools.partial(
        pl.kernel,
        out_type=jax.ShapeDtypeStruct((NPAD,), jnp.float32),
        mesh=mesh,
        scratch_types=[
            pltpu.VMEM((NPAD,), jnp.float32),
            pltpu.VMEM((EDGES_PER_SUB,), jnp.int32),
            pltpu.VMEM((EDGES_PER_SUB,), jnp.int32),
            pltpu.VMEM((NPAD,), jnp.float32),
            pltpu.VMEM((NSUB, NODES_PER_SUB), jnp.float32),
            pltpu.VMEM((NODES_PER_SUB,), jnp.float32),
            pltpu.VMEM((NODES_PER_SUB,), jnp.float32),
            pltpu.VMEM_SHARED((NSUB, NPAD), jnp.float32),
        ],
        compiler_params=pltpu.CompilerParams(needs_layout_passes=False),
    )
    def run(t_hbm, rb_hbm, src_hbm, dst_hbm, out_hbm,
            t_loc, src_loc, dst_loc, partial, tmp, rb_loc, out_loc, acc):
        sid = lax.axis_index("s")
        ebase = sid * EDGES_PER_SUB

        pltpu.sync_copy(t_hbm, t_loc)
        pltpu.sync_copy(src_hbm.at[pl.ds(ebase, EDGES_PER_SUB)], src_loc)
        pltpu.sync_copy(dst_hbm.at[pl.ds(ebase, EDGES_PER_SUB)], dst_loc)

        def zero_body(i, _):
            partial[pl.ds(i * 16, 16)] = jnp.zeros((16,), jnp.float32)
            return 0
        lax.fori_loop(0, NPAD // 16, zero_body, 0)

        def edge_body(i, _):
            off = i * 16
            s = src_loc[pl.ds(off, 16)]
            d = dst_loc[pl.ds(off, 16)]
            v = plsc.load_gather(t_loc, [s])
            plsc.addupdate_scatter(partial, [d], v)
            return 0
        lax.fori_loop(0, EDGE_VECS, edge_body, 0)

        pltpu.sync_copy(partial, acc.at[sid])
        plsc.subcore_barrier()

        nbase = sid * NODES_PER_SUB
        pltpu.sync_copy(acc.at[:, pl.ds(nbase, NODES_PER_SUB)], tmp)
        pltpu.sync_copy(rb_hbm.at[pl.ds(nbase, NODES_PER_SUB)], rb_loc)

        def red_body(j, _):
            col = j * 16
            v = tmp[0, pl.ds(col, 16)]
            for p in range(1, NSUB):
                v = v + tmp[p, pl.ds(col, 16)]
            v = v + rb_loc[pl.ds(col, 16)]
            out_loc[pl.ds(col, 16)] = v
            return 0
        lax.fori_loop(0, NODES_PER_SUB // 16, red_body, 0)

        pltpu.sync_copy(out_loc, out_hbm.at[pl.ds(nbase, NODES_PER_SUB)])

    return run(t, rb, src, dst)


def kernel(x, edges, rot, pos, batch, front_ptr, back_ptr, flat_new_idx,
           W_rel, b_rel, W_root):
    w1 = jnp.concatenate([W_rel[:D], W_root[:D]], axis=1)
    w2_top = jnp.concatenate([W_rel[D:], W_root[D:]], axis=1)
    w2_bias = jnp.concatenate(
        [jnp.zeros((1, 1), jnp.float32), b_rel.reshape(1, 1)], axis=1)
    w2 = jnp.concatenate([w2_top, w2_bias], axis=0)
    extra = jnp.concatenate(
        [pos, rot, jnp.ones((N, 1), jnp.float32)], axis=1)

    t, rb = _project(x, extra, w1, w2)
    t = t.reshape(NPAD)
    rb = rb.reshape(NPAD)

    out = _sc_scatter(t, rb, edges[0], edges[1])
    return out[:N].reshape(N, 1)
